# Initial kernel scaffold; baseline (speedup 1.0000x reference)
import jax
import jax.numpy as jnp
from jax.experimental import pallas as pl


def kernel(predictions, targets):
    raise NotImplementedError("write your pallas kernel here")


# speedup vs baseline: 1.4136x; 1.4136x over previous
import functools

import jax
import jax.numpy as jnp
from jax import lax
from jax.experimental import pallas as pl
from jax.experimental.pallas import tpu as pltpu
from jax.experimental.pallas import tpu_sc as plsc

_LANES = 16
_MIN_I32 = -2147483648


def _make_sc_numerators(n_cols, n_rows, num_workers, interpret=False):
    seglen = n_cols // _LANES
    rows_per = n_rows // num_workers
    tmask = seglen - 1
    tshift = seglen.bit_length() - 1
    radix = 256
    mu = (n_cols - 1) / 2.0

    mesh = plsc.VectorSubcoreMesh(core_axis_name="c", subcore_axis_name="s",
                                  num_cores=2, num_subcores=16)

    @functools.partial(
        pl.kernel,
        out_type=jax.ShapeDtypeStruct((n_rows * _LANES,), jnp.float32),
        mesh=mesh,
        scratch_types=[
            pltpu.VMEM((n_cols,), jnp.float32),
            pltpu.VMEM((n_cols,), jnp.float32),
            pltpu.VMEM((n_cols,), jnp.int32),
            pltpu.VMEM((n_cols,), jnp.int32),
            pltpu.VMEM((n_cols,), jnp.int32),
            pltpu.VMEM((n_cols,), jnp.int32),
            pltpu.VMEM((radix * _LANES,), jnp.int32),
            pltpu.VMEM((rows_per * _LANES,), jnp.float32),
        ],
        compiler_params=pltpu.CompilerParams(needs_layout_passes=False),
        interpret=interpret,
    )
    def sc_kernel(pred_hbm, targ_hbm, out_hbm,
                  bufp, buft, keys_a, keys_b, pay_a, pay_b, hist, nums):
        wid = lax.axis_index("s") * 2 + lax.axis_index("c")
        lane = lax.iota(jnp.int32, _LANES)
        zeros16 = jnp.zeros((_LANES,), jnp.int32)
        ones16 = jnp.ones((_LANES,), jnp.int32)

        def skew(i):
            t = i & tmask
            s = jnp.right_shift(i, tshift)
            return (t << 4) + ((s + t) & 15)

        def key_transpose(src_ref, dst_ref):
            def body(g, _):
                x = src_ref[pl.ds(g * _LANES, _LANES)]
                b = plsc.bitcast(x, jnp.int32)
                b = jnp.where(x == 0.0, 0, b)
                m = jnp.right_shift(b, 31)
                key = (b ^ (m & 0x7FFFFFFF)) ^ _MIN_I32
                i = g * _LANES + lane
                plsc.store_scatter(dst_ref, [skew(i)], key)
                return 0
            lax.fori_loop(0, seglen, body, 0)

        def radix_pass(shift, ksrc, psrc, kdst, pdst, gen_payload, last):
            def clr(d, _):
                hist[pl.ds(d * _LANES, _LANES)] = zeros16
                return 0
            lax.fori_loop(0, radix, clr, 0)

            def histo(t, _):
                k = ksrc[pl.ds(t * _LANES, _LANES)]
                digit = jnp.right_shift(k, shift) & 255
                s = (lane - t) & 15
                plsc.addupdate_scatter(hist, [(digit << 4) + s], ones16)
                return 0
            lax.fori_loop(0, seglen, histo, 0)

            def scan(d, tot):
                h = hist[pl.ds(d * _LANES, _LANES)]
                incl = plsc.cumsum(h)
                hist[pl.ds(d * _LANES, _LANES)] = incl - h + tot
                return tot + jnp.sum(h)
            lax.fori_loop(0, radix, scan, jnp.int32(0))

            def permute(t, _):
                k = ksrc[pl.ds(t * _LANES, _LANES)]
                digit = jnp.right_shift(k, shift) & 255
                s = (lane - t) & 15
                hidx = (digit << 4) + s
                pos = plsc.load_gather(hist, [hidx])
                plsc.store_scatter(hist, [hidx], pos + 1)
                if gen_payload:
                    payload = (s << tshift) + t
                else:
                    payload = psrc[pl.ds(t * _LANES, _LANES)]
                dest = skew(pos)
                if not last:
                    plsc.store_scatter(kdst, [dest], k)
                plsc.store_scatter(pdst, [dest], payload)
                return 0
            lax.fori_loop(0, seglen, permute, 0)

        def do_row(r, _):
            row = wid * rows_per + r
            pltpu.sync_copy(pred_hbm.at[row], bufp)
            pltpu.sync_copy(targ_hbm.at[row], buft)
            key_transpose(bufp, keys_a)
            radix_pass(0, keys_a, pay_a, keys_b, pay_b, True, False)
            radix_pass(8, keys_b, pay_b, keys_a, pay_a, False, False)
            radix_pass(16, keys_a, pay_a, keys_b, pay_b, False, False)
            radix_pass(24, keys_b, pay_b, keys_a, pay_a, False, True)
            key_transpose(buft, keys_b)

            def invert(t, _):
                p = pay_a[pl.ds(t * _LANES, _LANES)]
                s = (lane - t) & 15
                pos = (s << tshift) + t
                plsc.store_scatter(pay_b, [skew(p)], pos)
                return 0
            lax.fori_loop(0, seglen, invert, 0)
            radix_pass(0, keys_b, pay_b, keys_a, pay_a, False, False)
            radix_pass(8, keys_a, pay_a, keys_b, pay_b, False, False)
            radix_pass(16, keys_b, pay_b, keys_a, pay_a, False, False)
            radix_pass(24, keys_a, pay_a, keys_b, pay_b, False, True)

            def dot(t, acc):
                v = pay_b[pl.ds(t * _LANES, _LANES)]
                s = (lane - t) & 15
                m = (s << tshift) + t
                fm = m.astype(jnp.float32) - mu
                fv = v.astype(jnp.float32) - mu
                return acc + fm * fv
            acc = lax.fori_loop(0, seglen, dot, jnp.zeros((_LANES,), jnp.float32))
            nums[pl.ds(r * _LANES, _LANES)] = acc
            return 0

        lax.fori_loop(0, rows_per, do_row, 0)
        pltpu.sync_copy(
            nums, out_hbm.at[pl.ds(wid * rows_per * _LANES, rows_per * _LANES)])

    return sc_kernel


def _tc_reduce(nums, scale):
    n = nums.shape[0]
    x2d = nums.reshape(n // 128, 128)

    def body(x_ref, o_ref):
        o_ref[0, 0] = jnp.sum(x_ref[...]) * jnp.float32(scale)

    out = pl.pallas_call(
        body,
        out_shape=jax.ShapeDtypeStruct((1, 1), jnp.float32),
        in_specs=[pl.BlockSpec(memory_space=pltpu.VMEM)],
        out_specs=pl.BlockSpec(memory_space=pltpu.SMEM),
    )(x2d)
    return out[0, 0]


def kernel(predictions, targets):
    n_rows, n_cols = predictions.shape
    var = float(n_cols) * (float(n_cols) ** 2 - 1.0) / 12.0
    scale = -1.0 / ((var + 1e-8) * n_rows)
    sc = _make_sc_numerators(n_cols, n_rows, 32)
    nums = sc(predictions, targets)
    return _tc_reduce(nums, scale)


# speedup vs baseline: 1.8631x; 1.3180x over previous
import functools

import jax
import jax.numpy as jnp
from jax import lax
from jax.experimental import pallas as pl
from jax.experimental.pallas import tpu as pltpu
from jax.experimental.pallas import tpu_sc as plsc

_LANES = 16
_MIN_I32 = -2147483648


def _make_sc_numerators(n_cols, n_rows, num_workers, interpret=False):
    seglen = n_cols // _LANES
    rows_per = n_rows // num_workers
    tmask = seglen - 1
    tshift = seglen.bit_length() - 1
    radix = 256
    mu = (n_cols - 1) / 2.0

    mesh = plsc.VectorSubcoreMesh(core_axis_name="c", subcore_axis_name="s",
                                  num_cores=2, num_subcores=16)

    @functools.partial(
        pl.kernel,
        out_type=jax.ShapeDtypeStruct((n_rows * _LANES,), jnp.float32),
        mesh=mesh,
        scratch_types=[
            pltpu.VMEM((n_cols,), jnp.float32),
            pltpu.VMEM((n_cols,), jnp.float32),
            pltpu.VMEM((n_cols,), jnp.int32),
            pltpu.VMEM((n_cols,), jnp.int32),
            pltpu.VMEM((n_cols,), jnp.int32),
            pltpu.VMEM((n_cols,), jnp.int32),
            pltpu.VMEM((radix * _LANES,), jnp.int32),
            pltpu.VMEM((rows_per * _LANES,), jnp.float32),
        ],
        compiler_params=pltpu.CompilerParams(needs_layout_passes=False),
        interpret=interpret,
    )
    def sc_kernel(pred_hbm, targ_hbm, out_hbm,
                  bufp, buft, keys_a, keys_b, pay_a, pay_b, hist, nums):
        wid = lax.axis_index("s") * 2 + lax.axis_index("c")
        lane = lax.iota(jnp.int32, _LANES)
        zeros16 = jnp.zeros((_LANES,), jnp.int32)
        ones16 = jnp.ones((_LANES,), jnp.int32)

        def skew(i):
            t = i & tmask
            s = jnp.right_shift(i, tshift)
            return (t << 4) + ((s + t) & 15)

        def key_transpose(src_ref, dst_ref):
            def body(g, _):
                x = src_ref[pl.ds(g * _LANES, _LANES)]
                b = plsc.bitcast(x, jnp.int32)
                b = jnp.where(x == 0.0, 0, b)
                m = jnp.right_shift(b, 31)
                key = (b ^ (m & 0x7FFFFFFF)) ^ _MIN_I32
                i = g * _LANES + lane
                plsc.store_scatter(dst_ref, [skew(i)], key)
                return 0
            lax.fori_loop(0, seglen, body, 0, unroll=8)

        def radix_pass(shift, ksrc, psrc, kdst, pdst, gen_payload, last):
            def clr(d, _):
                hist[pl.ds(d * _LANES, _LANES)] = zeros16
                return 0
            lax.fori_loop(0, radix, clr, 0, unroll=8)

            def histo(t, _):
                k = ksrc[pl.ds(t * _LANES, _LANES)]
                digit = jnp.right_shift(k, shift) & 255
                s = (lane - t) & 15
                plsc.addupdate_scatter(hist, [(digit << 4) + s], ones16)
                return 0
            lax.fori_loop(0, seglen, histo, 0, unroll=8)

            def scan(d, tot):
                h = hist[pl.ds(d * _LANES, _LANES)]
                incl = plsc.cumsum(h)
                hist[pl.ds(d * _LANES, _LANES)] = incl - h + tot
                return tot + jnp.sum(h)
            lax.fori_loop(0, radix, scan, jnp.int32(0), unroll=4)

            def permute(t, _):
                k = ksrc[pl.ds(t * _LANES, _LANES)]
                digit = jnp.right_shift(k, shift) & 255
                s = (lane - t) & 15
                hidx = (digit << 4) + s
                pos = plsc.load_gather(hist, [hidx])
                plsc.store_scatter(hist, [hidx], pos + 1)
                if gen_payload:
                    payload = (s << tshift) + t
                else:
                    payload = psrc[pl.ds(t * _LANES, _LANES)]
                dest = skew(pos)
                if not last:
                    plsc.store_scatter(kdst, [dest], k)
                plsc.store_scatter(pdst, [dest], payload)
                return 0
            lax.fori_loop(0, seglen, permute, 0, unroll=4)

        def do_row(r, _):
            row = wid * rows_per + r
            pltpu.sync_copy(pred_hbm.at[row], bufp)
            pltpu.sync_copy(targ_hbm.at[row], buft)
            key_transpose(bufp, keys_a)
            radix_pass(0, keys_a, pay_a, keys_b, pay_b, True, False)
            radix_pass(8, keys_b, pay_b, keys_a, pay_a, False, False)
            radix_pass(16, keys_a, pay_a, keys_b, pay_b, False, False)
            radix_pass(24, keys_b, pay_b, keys_a, pay_a, False, True)
            key_transpose(buft, keys_b)

            def invert(t, _):
                p = pay_a[pl.ds(t * _LANES, _LANES)]
                s = (lane - t) & 15
                pos = (s << tshift) + t
                plsc.store_scatter(pay_b, [skew(p)], pos)
                return 0
            lax.fori_loop(0, seglen, invert, 0, unroll=8)
            radix_pass(0, keys_b, pay_b, keys_a, pay_a, False, False)
            radix_pass(8, keys_a, pay_a, keys_b, pay_b, False, False)
            radix_pass(16, keys_b, pay_b, keys_a, pay_a, False, False)
            radix_pass(24, keys_a, pay_a, keys_b, pay_b, False, True)

            def dot(t, acc):
                v = pay_b[pl.ds(t * _LANES, _LANES)]
                s = (lane - t) & 15
                m = (s << tshift) + t
                fm = m.astype(jnp.float32) - mu
                fv = v.astype(jnp.float32) - mu
                return acc + fm * fv
            acc = lax.fori_loop(0, seglen, dot, jnp.zeros((_LANES,), jnp.float32), unroll=8)
            nums[pl.ds(r * _LANES, _LANES)] = acc
            return 0

        lax.fori_loop(0, rows_per, do_row, 0)
        pltpu.sync_copy(
            nums, out_hbm.at[pl.ds(wid * rows_per * _LANES, rows_per * _LANES)])

    return sc_kernel


def _tc_reduce(nums, scale):
    n = nums.shape[0]
    x2d = nums.reshape(n // 128, 128)

    def body(x_ref, o_ref):
        o_ref[0, 0] = jnp.sum(x_ref[...]) * jnp.float32(scale)

    out = pl.pallas_call(
        body,
        out_shape=jax.ShapeDtypeStruct((1, 1), jnp.float32),
        in_specs=[pl.BlockSpec(memory_space=pltpu.VMEM)],
        out_specs=pl.BlockSpec(memory_space=pltpu.SMEM),
    )(x2d)
    return out[0, 0]


def kernel(predictions, targets):
    n_rows, n_cols = predictions.shape
    var = float(n_cols) * (float(n_cols) ** 2 - 1.0) / 12.0
    scale = -1.0 / ((var + 1e-8) * n_rows)
    sc = _make_sc_numerators(n_cols, n_rows, 32)
    nums = sc(predictions, targets)
    return _tc_reduce(nums, scale)
